# Initial kernel scaffold; baseline (speedup 1.0000x reference)
import jax
import jax.numpy as jnp
from jax.experimental import pallas as pl


def kernel(x, edge_index, W1, b1, W2, b2):
    raise NotImplementedError("write your pallas kernel here")


# speedup vs baseline: 105.6689x; 105.6689x over previous
import functools

import jax
import jax.numpy as jnp
from jax import lax
from jax.experimental import pallas as pl
from jax.experimental.pallas import tpu as pltpu
from jax.experimental.pallas import tpu_sc as plsc

N_NODES = 100000
N_EDGES = 6400000

NC = 2
NS = 16
NW = NC * NS

N_PAD = 100352
NPT = N_PAD // NS
PER_W = N_EDGES // NW
CHUNK = 2000
K_STEPS = PER_W // CHUNK

_mesh = plsc.VectorSubcoreMesh(core_axis_name="c", subcore_axis_name="s")


def _f32(shape):
    return jax.ShapeDtypeStruct(shape, jnp.float32)


@functools.partial(
    pl.kernel, mesh=_mesh,
    out_type=_f32((NC * N_PAD,)),
    scratch_types=[
        pltpu.VMEM((CHUNK,), jnp.int32),
        pltpu.VMEM((CHUNK,), jnp.float32),
        pltpu.VMEM((NPT,), jnp.float32),
        pltpu.VMEM_SHARED((N_PAD,), jnp.float32),
    ],
)
def _sc_degree(dst_hbm, out_hbm, idx_v, ones_v, zero_v, acc_sh):
    cid = lax.axis_index("c")
    sid = lax.axis_index("s")
    wid = sid * NC + cid

    @pl.loop(0, CHUNK, step=16)
    def _(i):
        ones_v[pl.ds(i, 16)] = jnp.full((16,), 1.0, jnp.float32)

    @pl.loop(0, NPT, step=16)
    def _(i):
        zero_v[pl.ds(i, 16)] = jnp.zeros((16,), jnp.float32)

    pltpu.sync_copy(zero_v, acc_sh.at[pl.ds(sid * NPT, NPT)])
    plsc.subcore_barrier()

    base = wid * PER_W

    @pl.loop(0, K_STEPS)
    def _(k):
        pltpu.sync_copy(dst_hbm.at[pl.ds(base + k * CHUNK, CHUNK)], idx_v)
        pltpu.sync_copy(ones_v, acc_sh.at[idx_v], add=True)

    plsc.subcore_barrier()
    pltpu.sync_copy(acc_sh.at[pl.ds(sid * NPT, NPT)],
                    out_hbm.at[pl.ds(cid * N_PAD + sid * NPT, NPT)])


CHUNK1 = 1000
K1_STEPS = PER_W // CHUNK1


@functools.partial(
    pl.kernel, mesh=_mesh,
    out_type=_f32((NC * N_PAD, 16)),
    compiler_params=pltpu.CompilerParams(use_tc_tiling_on_sc=False),
    scratch_types=[
        pltpu.VMEM((CHUNK1,), jnp.int32),
        pltpu.VMEM((CHUNK1,), jnp.int32),
        pltpu.VMEM((CHUNK1, 16), jnp.float32),
        pltpu.VMEM_SHARED((N_PAD, 16), jnp.float32),
    ],
)
def _sc_layer1(hs_hbm, src_hbm, dst_hbm, out_hbm, si_v, di_v, rows_v, acc_sh):
    cid = lax.axis_index("c")
    sid = lax.axis_index("s")
    wid = sid * NC + cid

    @pl.loop(0, CHUNK1)
    def _(i):
        rows_v[i, :] = jnp.zeros((16,), jnp.float32)

    @pl.loop(0, 6)
    def _(j):
        pltpu.sync_copy(rows_v, acc_sh.at[pl.ds(sid * NPT + j * CHUNK1,
                                                CHUNK1), :])
    pltpu.sync_copy(rows_v.at[pl.ds(0, NPT - 6 * CHUNK1), :],
                    acc_sh.at[pl.ds(sid * NPT + 6 * CHUNK1,
                                    NPT - 6 * CHUNK1), :])
    plsc.subcore_barrier()

    base = wid * PER_W

    @pl.loop(0, K1_STEPS)
    def _(k):
        pltpu.sync_copy(src_hbm.at[pl.ds(base + k * CHUNK1, CHUNK1)], si_v)
        pltpu.sync_copy(dst_hbm.at[pl.ds(base + k * CHUNK1, CHUNK1)], di_v)
        pltpu.sync_copy(hs_hbm.at[si_v], rows_v)
        pltpu.sync_copy(rows_v, acc_sh.at[di_v], add=True)

    plsc.subcore_barrier()
    pltpu.sync_copy(acc_sh.at[pl.ds(sid * NPT, NPT), :],
                    out_hbm.at[pl.ds(cid * N_PAD + sid * NPT, NPT), :])


@functools.partial(
    pl.kernel, mesh=_mesh,
    out_type=_f32((NC * N_PAD,)),
    scratch_types=[
        pltpu.VMEM((CHUNK,), jnp.int32),
        pltpu.VMEM((CHUNK,), jnp.int32),
        pltpu.VMEM((CHUNK,), jnp.float32),
        pltpu.VMEM((NPT,), jnp.float32),
        pltpu.VMEM_SHARED((N_PAD,), jnp.float32),
        pltpu.VMEM_SHARED((N_PAD,), jnp.float32),
    ],
)
def _sc_layer2(g_hbm, src_hbm, dst_hbm, out_hbm, si_v, di_v, val_v, zero_v,
               tab_sh, acc_sh):
    cid = lax.axis_index("c")
    sid = lax.axis_index("s")
    wid = sid * NC + cid

    @pl.loop(0, NPT, step=16)
    def _(i):
        zero_v[pl.ds(i, 16)] = jnp.zeros((16,), jnp.float32)

    pltpu.sync_copy(zero_v, acc_sh.at[pl.ds(sid * NPT, NPT)])
    pltpu.sync_copy(g_hbm.at[pl.ds(sid * NPT, NPT)],
                    tab_sh.at[pl.ds(sid * NPT, NPT)])
    plsc.subcore_barrier()

    base = wid * PER_W

    @pl.loop(0, K_STEPS)
    def _(k):
        pltpu.sync_copy(src_hbm.at[pl.ds(base + k * CHUNK, CHUNK)], si_v)
        pltpu.sync_copy(dst_hbm.at[pl.ds(base + k * CHUNK, CHUNK)], di_v)
        pltpu.sync_copy(tab_sh.at[si_v], val_v)
        pltpu.sync_copy(val_v, acc_sh.at[di_v], add=True)

    plsc.subcore_barrier()
    pltpu.sync_copy(acc_sh.at[pl.ds(sid * NPT, NPT)],
                    out_hbm.at[pl.ds(cid * N_PAD + sid * NPT, NPT)])


_ROWS = 2000
_GRID = N_NODES // _ROWS


def _row_spec(d):
    return pl.BlockSpec((_ROWS, d), lambda i: (i, 0))


def _full_spec(r, c):
    return pl.BlockSpec((r, c), lambda i: (0, 0))


def _tc_prep_body(d0_ref, d1_ref, x_ref, w1_ref, hs1_ref, dinv_ref):
    deg = d0_ref[...] + d1_ref[...] + 1.0
    dinv = lax.rsqrt(deg)
    h = jnp.dot(x_ref[...], w1_ref[...], preferred_element_type=jnp.float32)
    hs1_ref[...] = h * dinv
    dinv_ref[...] = dinv


def _tc_mid_body(p0_ref, p1_ref, hs1_ref, dinv_ref, b1_ref, w2_ref, g2_ref):
    dinv = dinv_ref[...]
    acc = p0_ref[...] + p1_ref[...] + hs1_ref[...]
    out1 = jnp.maximum(acc * dinv + b1_ref[...], 0.0)
    g2_ref[...] = jnp.dot(out1, w2_ref[...],
                          preferred_element_type=jnp.float32) * dinv


def _tc_final_body(p0_ref, p1_ref, g2_ref, dinv_ref, b2_ref, out_ref):
    out_ref[...] = ((p0_ref[...] + p1_ref[...] + g2_ref[...]) * dinv_ref[...]
                    + b2_ref[...])


def kernel(x, edge_index, W1, b1, W2, b2):
    src = edge_index[0].astype(jnp.int32)
    dst = edge_index[1].astype(jnp.int32)

    deg_p = _sc_degree(dst)
    d0 = deg_p[0:N_NODES, None]
    d1 = deg_p[N_PAD:N_PAD + N_NODES, None]

    hs1, dinv = pl.pallas_call(
        _tc_prep_body,
        grid=(_GRID,),
        in_specs=[_row_spec(1), _row_spec(1), _row_spec(5), _full_spec(5, 16)],
        out_specs=[_row_spec(16), _row_spec(1)],
        out_shape=[_f32((N_NODES, 16)), _f32((N_NODES, 1))],
    )(d0, d1, x, W1)

    acc1_p = _sc_layer1(hs1, src, dst)
    p0 = acc1_p[0:N_NODES]
    p1 = acc1_p[N_PAD:N_PAD + N_NODES]

    g2 = pl.pallas_call(
        _tc_mid_body,
        grid=(_GRID,),
        in_specs=[_row_spec(16), _row_spec(16), _row_spec(16), _row_spec(1),
                  _full_spec(1, 16), _full_spec(16, 1)],
        out_specs=_row_spec(1),
        out_shape=_f32((N_NODES, 1)),
    )(p0, p1, hs1, dinv, b1[None, :], W2)

    g2_pad = jnp.pad(g2[:, 0], (0, N_PAD - N_NODES))
    acc2_p = _sc_layer2(g2_pad, src, dst)
    q0 = acc2_p[0:N_NODES, None]
    q1 = acc2_p[N_PAD:N_PAD + N_NODES, None]

    out = pl.pallas_call(
        _tc_final_body,
        grid=(_GRID,),
        in_specs=[_row_spec(1), _row_spec(1), _row_spec(1), _row_spec(1),
                  _full_spec(1, 1)],
        out_specs=_row_spec(1),
        out_shape=_f32((N_NODES, 1)),
    )(q0, q1, g2, dinv, b2[None, :])
    return out
